# baseline (speedup 1.0000x reference)
import functools

import jax
import jax.numpy as jnp
from jax import lax
from jax.experimental import pallas as pl
from jax.experimental.pallas import tpu as pltpu
from jax.experimental.pallas import tpu_sc as plsc

N = 1024
E = 32768
NC = 2
NS = 16
EPT = E // NS
ROWS = EPT // 128
APW = (N * N) // NS


def _sc_body(g_hbm, ew_hbm, a_out, deg_out,
             a_sp, din_sp, dout_sp, sv2, dv2, wvf, xv, ov, zb, sem):
    c = lax.axis_index("c")
    s = lax.axis_index("s")

    z16 = jnp.zeros((16,), jnp.float32)
    o16 = jnp.ones((16,), jnp.float32)

    def zb_body(i, _):
        zb[pl.ds(i * 16, 16)] = z16
        return 0

    lax.fori_loop(0, 8192 // 16, zb_body, 0)

    def ov_body(i, _):
        ov[pl.ds(i * 16, 16)] = o16
        return 0

    lax.fori_loop(0, 128 // 16, ov_body, 0)

    @pl.when(s == 0)
    def _():
        pltpu.sync_copy(zb.at[pl.ds(0, N)], din_sp)
        pltpu.sync_copy(zb.at[pl.ds(0, N)], dout_sp)

    @pl.when(c == 0)
    def _():
        zcps = [
            pltpu.async_copy(zb, a_sp.at[pl.ds(s * APW + q * 8192, 8192)],
                             sem)
            for q in range(APW // 8192)
        ]
        pltpu.sync_copy(g_hbm.at[0, pl.ds(s * ROWS, ROWS), :], sv2)
        pltpu.sync_copy(g_hbm.at[1, pl.ds(s * ROWS, ROWS), :], dv2)
        pltpu.sync_copy(ew_hbm.at[pl.ds(s * ROWS, ROWS), :], wvf)

        for j in range(ROWS):
            def x_body(k, _):
                sl = pl.ds(k * 16, 16)
                r = dv2[j, sl]
                col = sv2[j, sl]
                xv[j, sl] = ((col >> 7) << 17) + ((r >> 3) << 10) \
                    + ((r & 7) << 7) + (col & 127)
                return 0
            lax.fori_loop(0, 128 // 16, x_body, 0)

        for cp in zcps:
            cp.wait()
        plsc.subcore_barrier()

        cps = []
        for j in range(ROWS):
            cps.append(pltpu.async_copy(
                wvf.at[j], a_sp.at[xv.at[j]], sem, add=True))
        for j in range(ROWS // 2):
            cps.append(pltpu.async_copy(ov, din_sp.at[dv2.at[j]], sem,
                                        add=True))
            cps.append(pltpu.async_copy(ov, dout_sp.at[sv2.at[j]], sem,
                                        add=True))
        for cp in cps:
            cp.wait()
        plsc.subcore_barrier()

        pltpu.sync_copy(a_sp.at[pl.ds(s * APW, APW)],
                        a_out.at[pl.ds(s * APW, APW)])

    @pl.when(c == 1)
    def _():
        pltpu.sync_copy(g_hbm.at[0, pl.ds(s * ROWS + 8, 8), :],
                        sv2.at[pl.ds(0, 8), :])
        pltpu.sync_copy(g_hbm.at[1, pl.ds(s * ROWS + 8, 8), :],
                        dv2.at[pl.ds(0, 8), :])
        plsc.subcore_barrier()
        cps = []
        for j in range(ROWS // 2):
            cps.append(pltpu.async_copy(ov, din_sp.at[dv2.at[j]], sem,
                                        add=True))
            cps.append(pltpu.async_copy(ov, dout_sp.at[sv2.at[j]], sem,
                                        add=True))
        for cp in cps:
            cp.wait()
        plsc.subcore_barrier()

    @pl.when(s == 0)
    def _():
        for kind in range(2):
            dsp = din_sp if kind == 0 else dout_sp
            for tc in range(8):
                pltpu.sync_copy(
                    dsp.at[pl.ds(tc * 128, 128)],
                    deg_out.at[pl.ds(tc * 1024 + (2 * c + kind) * 128, 128)])


def _sc_build(g3, ew2):
    mesh = plsc.VectorSubcoreMesh(core_axis_name="c", subcore_axis_name="s")
    f = pl.kernel(
        _sc_body,
        out_type=(
            jax.ShapeDtypeStruct((N * N,), jnp.float32),
            jax.ShapeDtypeStruct((8 * N,), jnp.float32),
        ),
        mesh=mesh,
        scratch_types=(
            pltpu.VMEM_SHARED((N * N,), jnp.float32),
            pltpu.VMEM_SHARED((N,), jnp.float32),
            pltpu.VMEM_SHARED((N,), jnp.float32),
            pltpu.VMEM((ROWS, 128), jnp.int32),
            pltpu.VMEM((ROWS, 128), jnp.int32),
            pltpu.VMEM((ROWS, 128), jnp.float32),
            pltpu.VMEM((ROWS, 128), jnp.int32),
            pltpu.VMEM((128,), jnp.float32),
            pltpu.VMEM((8192,), jnp.float32),
            pltpu.SemaphoreType.DMA,
        ),
    )
    return f(g3, ew2)


def _tc_body(ap_ref, deg_ref, feat_ref, w1_ref, b1_ref, wgt_ref,
             l1w_ref, l1b_ref, l2w_ref, l2b_ref, l3w_ref, l3b_ref, out_ref,
             m_s, t0_s, t1_s, ri_s):
    k = pl.program_id(0)
    dot = functools.partial(jnp.dot, preferred_element_type=jnp.float32)

    @pl.when(k == 0)
    def _():
        rit = jnp.concatenate(
            [lax.rsqrt(jnp.maximum(deg_ref[u * 8, :] + deg_ref[u * 8 + 2, :],
                                   1.0))[None, :]
             for u in range(8)], axis=1)
        ri_s[...] = lax.dot_general(
            rit, jnp.ones((1, 1), jnp.float32),
            (((0,), (0,)), ((), ())), preferred_element_type=jnp.float32)
        t0_s[...] = jnp.zeros_like(t0_s)
        t1_s[...] = jnp.zeros_like(t1_s)

    @pl.when(k < 8)
    def _():
        u = k
        ro_u = lax.rsqrt(jnp.maximum(
            deg_ref[u * 8 + 1, :] + deg_ref[u * 8 + 3, :], 1.0))[None, :]
        a_u = ap_ref[0].reshape(N, 128)
        m_u = a_u * ri_s[...] * ro_u
        m_s[:, pl.ds(u * 128, 128)] = m_u
        t0_s[...] += dot(m_u, feat_ref[0])

    @pl.when(k >= 8)
    def _():
        j = k - 8
        x1_j = jnp.maximum(
            dot(t0_s[...], w1_ref[0]) + b1_ref[pl.ds(j * 256, 256)][None, :],
            0.0)
        t1_s[...] += dot(x1_j.astype(jnp.bfloat16), wgt_ref[0])

    @pl.when(k == 15)
    def _():
        x2 = jnp.maximum(
            dot(m_s[...].astype(jnp.bfloat16), t1_s[...].astype(jnp.bfloat16)),
            0.0)
        x3 = jnp.maximum(dot(x2, l1w_ref[...]) + l1b_ref[...][None, :], 0.0)
        x4 = jnp.maximum(dot(x3, l2w_ref[...]) + l2b_ref[...][None, :], 0.0)
        out_ref[...] = dot(x4, l3w_ref[...]) + l3b_ref[...][None, :]


def kernel(g, features, weight, edge_weight, W1, b1, lin1_W, lin1_b,
           lin2_W, lin2_b, lin3_W, lin3_b):
    a_flat, deg_flat = _sc_build(g.reshape(2, E // 128, 128),
                                 edge_weight.reshape(E // 128, 128))
    ap = a_flat.reshape(8, 128, 8, 128)
    degs = deg_flat.reshape(64, 128)
    wbf = weight.astype(jnp.bfloat16).reshape(8, 256, N)
    feat3 = features.reshape(8, 128, -1)
    w1c = W1.reshape(256, 8, 256).transpose(1, 0, 2)

    out = pl.pallas_call(
        _tc_body,
        grid=(16,),
        in_specs=[
            pl.BlockSpec((1, 128, 8, 128),
                         lambda k: (jnp.minimum(k, 7), 0, 0, 0)),
            pl.BlockSpec((64, 128), lambda k: (0, 0)),
            pl.BlockSpec((1, 128, 256), lambda k: (jnp.minimum(k, 7), 0, 0)),
            pl.BlockSpec((1, 256, 256),
                         lambda k: (jnp.maximum(k - 8, 0), 0, 0)),
            pl.BlockSpec((2 * N,), lambda k: (0,)),
            pl.BlockSpec((1, 256, N), lambda k: (jnp.maximum(k - 8, 0), 0, 0)),
            pl.BlockSpec((N, 64), lambda k: (0, 0)),
            pl.BlockSpec((64,), lambda k: (0,)),
            pl.BlockSpec((64, 16), lambda k: (0, 0)),
            pl.BlockSpec((16,), lambda k: (0,)),
            pl.BlockSpec((16, 16), lambda k: (0, 0)),
            pl.BlockSpec((16,), lambda k: (0,)),
        ],
        out_specs=pl.BlockSpec((N, 16), lambda k: (0, 0)),
        out_shape=jax.ShapeDtypeStruct((N, 16), jnp.float32),
        scratch_shapes=[
            pltpu.VMEM((N, N), jnp.float32),
            pltpu.VMEM((N, 256), jnp.float32),
            pltpu.VMEM((N, N), jnp.float32),
            pltpu.VMEM((N, 1), jnp.float32),
        ],
    )(ap, degs, feat3, w1c, b1, wbf,
      lin1_W, lin1_b, lin2_W, lin2_b, lin3_W, lin3_b)
    return out


# speedup vs baseline: 1.1192x; 1.1192x over previous
import functools

import jax
import jax.numpy as jnp
from jax import lax
from jax.experimental import pallas as pl
from jax.experimental.pallas import tpu as pltpu
from jax.experimental.pallas import tpu_sc as plsc

N = 1024
E = 32768
NC = 2
NS = 16
NW = NC * NS
EPW = E // NW
ROWS = EPW // 128
APW = (N * N) // NS


def _sc_body(g_hbm, ew_hbm, a0_out, a1_out, deg_out,
             a_sp, dstage, sv2, dv2, wvf, xv, dloc, red, redout, zb, sem):
    c = lax.axis_index("c")
    s = lax.axis_index("s")
    wid = s * NC + c

    z16 = jnp.zeros((16,), jnp.float32)
    o16 = jnp.ones((16,), jnp.float32)

    def zb_body(i, _):
        zb[pl.ds(i * 16, 16)] = z16
        return 0

    lax.fori_loop(0, 8192 // 16, zb_body, 0)

    def dl_body(i, _):
        dloc[pl.ds(i * 16, 16)] = z16
        return 0

    lax.fori_loop(0, 2048 // 16, dl_body, 0)

    zcps = [
        pltpu.async_copy(zb, a_sp.at[pl.ds(s * APW + q * 8192, 8192)], sem)
        for q in range(APW // 8192)
    ]

    pltpu.sync_copy(g_hbm.at[0, pl.ds(wid * ROWS, ROWS), :], sv2)
    pltpu.sync_copy(g_hbm.at[1, pl.ds(wid * ROWS, ROWS), :], dv2)
    pltpu.sync_copy(ew_hbm.at[pl.ds(wid * ROWS, ROWS), :], wvf)

    for j in range(ROWS):
        for k in range(128 // 16):
            sl = pl.ds(k * 16, 16)
            r = dv2[j, sl]
            col = sv2[j, sl]
            xv[j, sl] = ((col >> 7) << 17) + ((r >> 3) << 10) \
                + ((r & 7) << 7) + (col & 127)
            plsc.addupdate_scatter(dloc, [r], o16)
            plsc.addupdate_scatter(dloc, [col + 1024], o16)

    pltpu.sync_copy(dloc, dstage.at[s])

    for cp in zcps:
        cp.wait()
    plsc.subcore_barrier()

    cps = [pltpu.async_copy(wvf.at[j], a_sp.at[xv.at[j]], sem, add=True)
           for j in range(ROWS)]

    pltpu.sync_copy(dstage.at[:, pl.ds(s * 128, 128)], red)

    def r_body(k, _):
        sl = pl.ds(k * 16, 16)
        acc = red[0, sl]
        for t in range(1, NS):
            acc = acc + red[t, sl]
        redout[sl] = acc
        return 0

    lax.fori_loop(0, 128 // 16, r_body, 0)

    pltpu.sync_copy(
        redout,
        deg_out.at[pl.ds((s & 7) * 1024 + (2 * c + (s >> 3)) * 128, 128)])

    for cp in cps:
        cp.wait()
    plsc.subcore_barrier()

    @pl.when(c == 0)
    def _():
        pltpu.sync_copy(a_sp.at[pl.ds(s * APW, APW)],
                        a0_out.at[pl.ds(s * APW, APW)])

    @pl.when(c == 1)
    def _():
        pltpu.sync_copy(a_sp.at[pl.ds(s * APW, APW)],
                        a1_out.at[pl.ds(s * APW, APW)])


def _sc_build(g3, ew2):
    mesh = plsc.VectorSubcoreMesh(core_axis_name="c", subcore_axis_name="s")
    f = pl.kernel(
        _sc_body,
        out_type=(
            jax.ShapeDtypeStruct((N * N,), jnp.float32),
            jax.ShapeDtypeStruct((N * N,), jnp.float32),
            jax.ShapeDtypeStruct((8 * N,), jnp.float32),
        ),
        mesh=mesh,
        compiler_params=pltpu.CompilerParams(needs_layout_passes=False),
        scratch_types=(
            pltpu.VMEM_SHARED((N * N,), jnp.float32),
            pltpu.VMEM_SHARED((NS, 2048), jnp.float32),
            pltpu.VMEM((ROWS, 128), jnp.int32),
            pltpu.VMEM((ROWS, 128), jnp.int32),
            pltpu.VMEM((ROWS, 128), jnp.float32),
            pltpu.VMEM((ROWS, 128), jnp.int32),
            pltpu.VMEM((2048,), jnp.float32),
            pltpu.VMEM((NS, 128), jnp.float32),
            pltpu.VMEM((128,), jnp.float32),
            pltpu.VMEM((8192,), jnp.float32),
            pltpu.SemaphoreType.DMA,
        ),
    )
    return f(g3, ew2)


def _tc_body(a0_ref, a1_ref, deg_ref, feat_ref, w1_ref, b1_ref, wgt_ref,
             l1w_ref, l1b_ref, l2w_ref, l2b_ref, l3w_ref, l3b_ref, out_ref,
             m_s, t0_s, t1_s, ri_s):
    k = pl.program_id(0)
    dot = functools.partial(jnp.dot, preferred_element_type=jnp.float32)

    @pl.when(k == 0)
    def _():
        rit = jnp.concatenate(
            [lax.rsqrt(jnp.maximum(deg_ref[u * 8, :] + deg_ref[u * 8 + 2, :],
                                   1.0))[None, :]
             for u in range(8)], axis=1)
        ri_s[...] = lax.dot_general(
            rit, jnp.ones((1, 1), jnp.float32),
            (((0,), (0,)), ((), ())), preferred_element_type=jnp.float32)
        t0_s[...] = jnp.zeros_like(t0_s)
        t1_s[...] = jnp.zeros_like(t1_s)

    @pl.when(k < 8)
    def _():
        u = k
        ro_u = lax.rsqrt(jnp.maximum(
            deg_ref[u * 8 + 1, :] + deg_ref[u * 8 + 3, :], 1.0))[None, :]
        a_u = (a0_ref[0] + a1_ref[0]).reshape(N, 128)
        m_u = a_u * ri_s[...] * ro_u
        m_s[:, pl.ds(u * 128, 128)] = m_u
        t0_s[...] += dot(m_u, feat_ref[0])

    @pl.when(k >= 8)
    def _():
        j = k - 8
        x1_j = jnp.maximum(
            dot(t0_s[...], w1_ref[0]) + b1_ref[pl.ds(j * 256, 256)][None, :],
            0.0)
        t1_s[...] += dot(x1_j.astype(jnp.bfloat16), wgt_ref[0])

    @pl.when(k == 15)
    def _():
        x2 = jnp.maximum(
            dot(m_s[...].astype(jnp.bfloat16), t1_s[...].astype(jnp.bfloat16)),
            0.0)
        x3 = jnp.maximum(dot(x2, l1w_ref[...]) + l1b_ref[...][None, :], 0.0)
        x4 = jnp.maximum(dot(x3, l2w_ref[...]) + l2b_ref[...][None, :], 0.0)
        out_ref[...] = dot(x4, l3w_ref[...]) + l3b_ref[...][None, :]


def kernel(g, features, weight, edge_weight, W1, b1, lin1_W, lin1_b,
           lin2_W, lin2_b, lin3_W, lin3_b):
    a0_flat, a1_flat, deg_flat = _sc_build(
        g.reshape(2, E // 128, 128), edge_weight.reshape(E // 128, 128))
    ap0 = a0_flat.reshape(8, 128, 8, 128)
    ap1 = a1_flat.reshape(8, 128, 8, 128)
    degs = deg_flat.reshape(64, 128)
    wbf = weight.astype(jnp.bfloat16).reshape(8, 256, N)
    feat3 = features.reshape(8, 128, -1)
    w1c = W1.reshape(256, 8, 256).transpose(1, 0, 2)

    blk_u = pl.BlockSpec((1, 128, 8, 128),
                         lambda k: (jnp.minimum(k, 7), 0, 0, 0))
    out = pl.pallas_call(
        _tc_body,
        grid=(16,),
        in_specs=[
            blk_u,
            blk_u,
            pl.BlockSpec((64, 128), lambda k: (0, 0)),
            pl.BlockSpec((1, 128, 256), lambda k: (jnp.minimum(k, 7), 0, 0)),
            pl.BlockSpec((1, 256, 256),
                         lambda k: (jnp.maximum(k - 8, 0), 0, 0)),
            pl.BlockSpec((2 * N,), lambda k: (0,)),
            pl.BlockSpec((1, 256, N), lambda k: (jnp.maximum(k - 8, 0), 0, 0)),
            pl.BlockSpec((N, 64), lambda k: (0, 0)),
            pl.BlockSpec((64,), lambda k: (0,)),
            pl.BlockSpec((64, 16), lambda k: (0, 0)),
            pl.BlockSpec((16,), lambda k: (0,)),
            pl.BlockSpec((16, 16), lambda k: (0, 0)),
            pl.BlockSpec((16,), lambda k: (0,)),
        ],
        out_specs=pl.BlockSpec((N, 16), lambda k: (0, 0)),
        out_shape=jax.ShapeDtypeStruct((N, 16), jnp.float32),
        scratch_shapes=[
            pltpu.VMEM((N, N), jnp.float32),
            pltpu.VMEM((N, 256), jnp.float32),
            pltpu.VMEM((N, N), jnp.float32),
            pltpu.VMEM((N, 1), jnp.float32),
        ],
    )(ap0, ap1, degs, feat3, w1c, b1, wbf,
      lin1_W, lin1_b, lin2_W, lin2_b, lin3_W, lin3_b)
    return out


# speedup vs baseline: 1.1308x; 1.0103x over previous
import functools

import jax
import jax.numpy as jnp
from jax import lax
from jax.experimental import pallas as pl
from jax.experimental.pallas import tpu as pltpu
from jax.experimental.pallas import tpu_sc as plsc

N = 1024
E = 32768
NC = 2
NS = 16
HR = N // NC
EPT = E // NS
ROWS = EPT // 128
HAPW = (HR * N) // NS


def _sc_body(g_hbm, ew_hbm, a0_out, a1_out, deg_out,
             a_sp, dstage, sv2, dv2, wvf, wsel, xv, dloc, red, redout, zb,
             sem):
    c = lax.axis_index("c")
    s = lax.axis_index("s")

    z16 = jnp.zeros((16,), jnp.float32)
    o16 = jnp.ones((16,), jnp.float32)

    def zb_body(i, _):
        zb[pl.ds(i * 16, 16)] = z16
        return 0

    lax.fori_loop(0, 8192 // 16, zb_body, 0)

    def dl_body(i, _):
        dloc[pl.ds(i * 16, 16)] = z16
        return 0

    lax.fori_loop(0, 2048 // 16, dl_body, 0)

    zcps = [
        pltpu.async_copy(zb, a_sp.at[pl.ds(s * HAPW + q * 8192, 8192)], sem)
        for q in range(HAPW // 8192)
    ]

    pltpu.sync_copy(g_hbm.at[0, pl.ds(s * ROWS, ROWS), :], sv2)
    pltpu.sync_copy(g_hbm.at[1, pl.ds(s * ROWS, ROWS), :], dv2)
    pltpu.sync_copy(ew_hbm.at[pl.ds(s * ROWS, ROWS), :], wvf)

    for j in range(ROWS):
        for k in range(128 // 16):
            sl = pl.ds(k * 16, 16)
            r = dv2[j, sl]
            col = sv2[j, sl]
            keep = (r >> 9) == c
            ksel = jnp.where(keep, o16, z16)
            rl = r & (HR - 1)
            xv[j, sl] = ((col >> 7) << 16) + ((rl >> 3) << 10) \
                + ((rl & 7) << 7) + (col & 127)
            wsel[j, sl] = jnp.where(keep, wvf[j, sl], z16)
            plsc.addupdate_scatter(dloc, [r], ksel)
            plsc.addupdate_scatter(dloc, [col + 1024], ksel)

    pltpu.sync_copy(dloc, dstage.at[s])

    for cp in zcps:
        cp.wait()
    plsc.subcore_barrier()

    cps = [pltpu.async_copy(wsel.at[j], a_sp.at[xv.at[j]], sem, add=True)
           for j in range(ROWS)]

    pltpu.sync_copy(dstage.at[:, pl.ds(s * 128, 128)], red)

    def r_body(k, _):
        sl = pl.ds(k * 16, 16)
        acc = red[0, sl]
        for t in range(1, NS):
            acc = acc + red[t, sl]
        redout[sl] = acc
        return 0

    lax.fori_loop(0, 128 // 16, r_body, 0)

    pltpu.sync_copy(
        redout,
        deg_out.at[pl.ds((s & 7) * 1024 + (2 * c + (s >> 3)) * 128, 128)])

    for cp in cps:
        cp.wait()
    plsc.subcore_barrier()

    @pl.when(c == 0)
    def _():
        pltpu.sync_copy(a_sp.at[pl.ds(s * HAPW, HAPW)],
                        a0_out.at[pl.ds(s * HAPW, HAPW)])

    @pl.when(c == 1)
    def _():
        pltpu.sync_copy(a_sp.at[pl.ds(s * HAPW, HAPW)],
                        a1_out.at[pl.ds(s * HAPW, HAPW)])


def _sc_build(g3, ew2):
    mesh = plsc.VectorSubcoreMesh(core_axis_name="c", subcore_axis_name="s")
    f = pl.kernel(
        _sc_body,
        out_type=(
            jax.ShapeDtypeStruct((HR * N,), jnp.float32),
            jax.ShapeDtypeStruct((HR * N,), jnp.float32),
            jax.ShapeDtypeStruct((8 * N,), jnp.float32),
        ),
        mesh=mesh,
        compiler_params=pltpu.CompilerParams(needs_layout_passes=False),
        scratch_types=(
            pltpu.VMEM_SHARED((HR * N,), jnp.float32),
            pltpu.VMEM_SHARED((NS, 2048), jnp.float32),
            pltpu.VMEM((ROWS, 128), jnp.int32),
            pltpu.VMEM((ROWS, 128), jnp.int32),
            pltpu.VMEM((ROWS, 128), jnp.float32),
            pltpu.VMEM((ROWS, 128), jnp.float32),
            pltpu.VMEM((ROWS, 128), jnp.int32),
            pltpu.VMEM((2048,), jnp.float32),
            pltpu.VMEM((NS, 128), jnp.float32),
            pltpu.VMEM((128,), jnp.float32),
            pltpu.VMEM((8192,), jnp.float32),
            pltpu.SemaphoreType.DMA,
        ),
    )
    return f(g3, ew2)


def _tc_body(a0_ref, a1_ref, deg_ref, feat_ref, w1_ref, b1_ref, wgt_ref,
             l1w_ref, l1b_ref, l2w_ref, l2b_ref, l3w_ref, l3b_ref, out_ref,
             m_s, t0_s, t1_s, ri_s):
    k = pl.program_id(0)
    dot = functools.partial(jnp.dot, preferred_element_type=jnp.float32)

    @pl.when(k == 0)
    def _():
        rit = jnp.concatenate(
            [lax.rsqrt(jnp.maximum(deg_ref[u * 8, :] + deg_ref[u * 8 + 2, :],
                                   1.0))[None, :]
             for u in range(8)], axis=1)
        ri_s[...] = lax.dot_general(
            rit, jnp.ones((1, 1), jnp.float32),
            (((0,), (0,)), ((), ())), preferred_element_type=jnp.float32)
        t0_s[...] = jnp.zeros_like(t0_s)
        t1_s[...] = jnp.zeros_like(t1_s)

    @pl.when(k < 8)
    def _():
        u = k
        ro_u = lax.rsqrt(jnp.maximum(
            deg_ref[u * 8 + 1, :] + deg_ref[u * 8 + 3, :], 1.0))[None, :]
        m_top = a0_ref[0].reshape(HR, 128) * ri_s[:HR] * ro_u
        m_bot = a1_ref[0].reshape(HR, 128) * ri_s[HR:] * ro_u
        m_s[:HR, pl.ds(u * 128, 128)] = m_top
        m_s[HR:, pl.ds(u * 128, 128)] = m_bot
        fu = feat_ref[0].astype(jnp.bfloat16)
        t0_s[:HR] += dot(m_top.astype(jnp.bfloat16), fu)
        t0_s[HR:] += dot(m_bot.astype(jnp.bfloat16), fu)

    @pl.when(k >= 8)
    def _():
        j = k - 8
        x1_j = jnp.maximum(
            dot(t0_s[...].astype(jnp.bfloat16), w1_ref[0])
            + b1_ref[pl.ds(j * 256, 256)][None, :],
            0.0)
        t1_s[...] += dot(x1_j.astype(jnp.bfloat16), wgt_ref[0])

    @pl.when(k == 15)
    def _():
        x2 = jnp.maximum(
            dot(m_s[...].astype(jnp.bfloat16), t1_s[...].astype(jnp.bfloat16)),
            0.0)
        x3 = jnp.maximum(dot(x2, l1w_ref[...]) + l1b_ref[...][None, :], 0.0)
        x4 = jnp.maximum(dot(x3, l2w_ref[...]) + l2b_ref[...][None, :], 0.0)
        out_ref[...] = dot(x4, l3w_ref[...]) + l3b_ref[...][None, :]


def kernel(g, features, weight, edge_weight, W1, b1, lin1_W, lin1_b,
           lin2_W, lin2_b, lin3_W, lin3_b):
    a0_flat, a1_flat, deg_flat = _sc_build(
        g.reshape(2, E // 128, 128), edge_weight.reshape(E // 128, 128))
    ap0 = a0_flat.reshape(8, 64, 8, 128)
    ap1 = a1_flat.reshape(8, 64, 8, 128)
    degs = deg_flat.reshape(64, 128)
    wbf = weight.astype(jnp.bfloat16).reshape(8, 256, N)
    feat3 = features.reshape(8, 128, -1)
    w1c = W1.astype(jnp.bfloat16).reshape(256, 8, 256).transpose(1, 0, 2)

    blk_u = pl.BlockSpec((1, 64, 8, 128),
                         lambda k: (jnp.minimum(k, 7), 0, 0, 0))
    out = pl.pallas_call(
        _tc_body,
        grid=(16,),
        in_specs=[
            blk_u,
            blk_u,
            pl.BlockSpec((64, 128), lambda k: (0, 0)),
            pl.BlockSpec((1, 128, 256), lambda k: (jnp.minimum(k, 7), 0, 0)),
            pl.BlockSpec((1, 256, 256),
                         lambda k: (jnp.maximum(k - 8, 0), 0, 0)),
            pl.BlockSpec((2 * N,), lambda k: (0,)),
            pl.BlockSpec((1, 256, N), lambda k: (jnp.maximum(k - 8, 0), 0, 0)),
            pl.BlockSpec((N, 64), lambda k: (0, 0)),
            pl.BlockSpec((64,), lambda k: (0,)),
            pl.BlockSpec((64, 16), lambda k: (0, 0)),
            pl.BlockSpec((16,), lambda k: (0,)),
            pl.BlockSpec((16, 16), lambda k: (0, 0)),
            pl.BlockSpec((16,), lambda k: (0,)),
        ],
        out_specs=pl.BlockSpec((N, 16), lambda k: (0, 0)),
        out_shape=jax.ShapeDtypeStruct((N, 16), jnp.float32),
        scratch_shapes=[
            pltpu.VMEM((N, N), jnp.float32),
            pltpu.VMEM((N, 256), jnp.float32),
            pltpu.VMEM((N, N), jnp.float32),
            pltpu.VMEM((N, 1), jnp.float32),
        ],
    )(ap0, ap1, degs, feat3, w1c, b1, wbf,
      lin1_W, lin1_b, lin2_W, lin2_b, lin3_W, lin3_b)
    return out


# speedup vs baseline: 1.1877x; 1.0504x over previous
import functools

import jax
import jax.numpy as jnp
from jax import lax
from jax.experimental import pallas as pl
from jax.experimental.pallas import tpu as pltpu
from jax.experimental.pallas import tpu_sc as plsc

N = 1024
E = 32768
NC = 2
NS = 16
HR = N // NC
EPT = E // NS
ROWS = EPT // 128
HAPW = (HR * N) // NS


def _sc_body(g_hbm, ew_hbm, a0_out, a1_out, deg_out,
             a_sp, dstage, sv2, dv2, wvf, wsel, xv, dloc, red, redout, zb,
             sem):
    c = lax.axis_index("c")
    s = lax.axis_index("s")

    z16 = jnp.zeros((16,), jnp.float32)
    o16 = jnp.ones((16,), jnp.float32)

    def zb_body(i, _):
        zb[pl.ds(i * 16, 16)] = z16
        return 0

    lax.fori_loop(0, 8192 // 16, zb_body, 0)

    def dl_body(i, _):
        dloc[pl.ds(i * 16, 16)] = z16
        return 0

    lax.fori_loop(0, 2048 // 16, dl_body, 0)

    zcps = [
        pltpu.async_copy(zb, a_sp.at[pl.ds(s * HAPW + q * 8192, 8192)], sem)
        for q in range(HAPW // 8192)
    ]

    pltpu.sync_copy(g_hbm.at[0, pl.ds(s * ROWS, ROWS), :], sv2)
    pltpu.sync_copy(g_hbm.at[1, pl.ds(s * ROWS, ROWS), :], dv2)
    pltpu.sync_copy(ew_hbm.at[pl.ds(s * ROWS, ROWS), :], wvf)

    for j in range(ROWS):
        for k in range(128 // 16):
            sl = pl.ds(k * 16, 16)
            r = dv2[j, sl]
            col = sv2[j, sl]
            keep = (r >> 9) == c
            ksel = jnp.where(keep, o16, z16)
            rl = r & (HR - 1)
            xv[j, sl] = ((col >> 7) << 16) + ((rl >> 3) << 10) \
                + ((rl & 7) << 7) + (col & 127)
            wsel[j, sl] = jnp.where(keep, wvf[j, sl], z16)
            plsc.addupdate_scatter(dloc, [r], ksel)
            plsc.addupdate_scatter(dloc, [col + 1024], ksel)

    pltpu.sync_copy(dloc, dstage.at[s])

    for cp in zcps:
        cp.wait()
    plsc.subcore_barrier()

    cps = [pltpu.async_copy(wsel.at[j], a_sp.at[xv.at[j]], sem, add=True)
           for j in range(ROWS)]

    pltpu.sync_copy(dstage.at[:, pl.ds(s * 128, 128)], red)

    def r_body(k, _):
        sl = pl.ds(k * 16, 16)
        acc = red[0, sl]
        for t in range(1, NS):
            acc = acc + red[t, sl]
        redout[sl] = acc
        return 0

    lax.fori_loop(0, 128 // 16, r_body, 0)

    pltpu.sync_copy(
        redout,
        deg_out.at[pl.ds((s & 7) * 1024 + (2 * c + (s >> 3)) * 128, 128)])

    for cp in cps:
        cp.wait()
    plsc.subcore_barrier()

    @pl.when(c == 0)
    def _():
        pltpu.sync_copy(a_sp.at[pl.ds(s * HAPW, HAPW)],
                        a0_out.at[pl.ds(s * HAPW, HAPW)])

    @pl.when(c == 1)
    def _():
        pltpu.sync_copy(a_sp.at[pl.ds(s * HAPW, HAPW)],
                        a1_out.at[pl.ds(s * HAPW, HAPW)])


def _sc_build(g3, ew2):
    mesh = plsc.VectorSubcoreMesh(core_axis_name="c", subcore_axis_name="s")
    f = pl.kernel(
        _sc_body,
        out_type=(
            jax.ShapeDtypeStruct((HR * N,), jnp.float32),
            jax.ShapeDtypeStruct((HR * N,), jnp.float32),
            jax.ShapeDtypeStruct((8 * N,), jnp.float32),
        ),
        mesh=mesh,
        compiler_params=pltpu.CompilerParams(needs_layout_passes=False),
        scratch_types=(
            pltpu.VMEM_SHARED((HR * N,), jnp.float32),
            pltpu.VMEM_SHARED((NS, 2048), jnp.float32),
            pltpu.VMEM((ROWS, 128), jnp.int32),
            pltpu.VMEM((ROWS, 128), jnp.int32),
            pltpu.VMEM((ROWS, 128), jnp.float32),
            pltpu.VMEM((ROWS, 128), jnp.float32),
            pltpu.VMEM((ROWS, 128), jnp.int32),
            pltpu.VMEM((2048,), jnp.float32),
            pltpu.VMEM((NS, 128), jnp.float32),
            pltpu.VMEM((128,), jnp.float32),
            pltpu.VMEM((8192,), jnp.float32),
            pltpu.SemaphoreType.DMA,
        ),
    )
    return f(g3, ew2)


def _tc_body(a0_hbm, a1_hbm, deg_ref, feat_hbm, w1_hbm, b1_ref, wgt_hbm,
             l1w_ref, l1b_ref, l2w_ref, l2b_ref, l3w_ref, l3b_ref, out_ref,
             a0_s, a1_s, feat_s, w1_s, wgt_s, sems):
    dot = functools.partial(jnp.dot, preferred_element_type=jnp.float32)

    cp0 = pltpu.async_copy(a0_hbm, a0_s, sems.at[0])
    cp1 = pltpu.async_copy(a1_hbm, a1_s, sems.at[1])
    cp2 = pltpu.async_copy(feat_hbm, feat_s, sems.at[2])
    cp3 = pltpu.async_copy(w1_hbm, w1_s, sems.at[3])
    cp4 = pltpu.async_copy(wgt_hbm.at[pl.ds(0, 4)], wgt_s.at[pl.ds(0, 4)],
                           sems.at[4])
    cp5 = pltpu.async_copy(wgt_hbm.at[pl.ds(4, 4)], wgt_s.at[pl.ds(4, 4)],
                           sems.at[5])

    rit = jnp.concatenate(
        [lax.rsqrt(jnp.maximum(deg_ref[u * 8, :] + deg_ref[u * 8 + 2, :],
                               1.0))[None, :]
         for u in range(8)], axis=1)
    ri = lax.dot_general(
        rit, jnp.ones((1, 1), jnp.float32),
        (((0,), (0,)), ((), ())), preferred_element_type=jnp.float32)
    ros = [lax.rsqrt(jnp.maximum(
        deg_ref[u * 8 + 1, :] + deg_ref[u * 8 + 3, :], 1.0))[None, :]
        for u in range(8)]

    cp0.wait()
    cp1.wait()
    cp2.wait()
    m_us = []
    t0_top = jnp.zeros((HR, 256), jnp.float32)
    t0_bot = jnp.zeros((HR, 256), jnp.float32)
    for u in range(8):
        m_top = a0_s[u].reshape(HR, 128) * ri[:HR] * ros[u]
        m_bot = a1_s[u].reshape(HR, 128) * ri[HR:] * ros[u]
        m_us.append((m_top.astype(jnp.bfloat16), m_bot.astype(jnp.bfloat16)))
        fu = feat_s[u].astype(jnp.bfloat16)
        t0_top = t0_top + dot(m_us[u][0], fu)
        t0_bot = t0_bot + dot(m_us[u][1], fu)
    t0 = jnp.concatenate([t0_top, t0_bot], axis=0).astype(jnp.bfloat16)

    cp3.wait()
    cp4.wait()
    cp5.wait()
    t1 = jnp.zeros((N, N), jnp.float32)
    for j in range(8):
        x1_j = jnp.maximum(
            dot(t0, w1_s[j]) + b1_ref[pl.ds(j * 256, 256)][None, :], 0.0)
        t1 = t1 + dot(x1_j.astype(jnp.bfloat16), wgt_s[j])
    t1 = t1.astype(jnp.bfloat16)

    x2_top = jnp.zeros((HR, N), jnp.float32)
    x2_bot = jnp.zeros((HR, N), jnp.float32)
    for u in range(8):
        t1_u = t1[u * 128:(u + 1) * 128, :]
        x2_top = x2_top + dot(m_us[u][0], t1_u)
        x2_bot = x2_bot + dot(m_us[u][1], t1_u)
    x2 = jnp.maximum(jnp.concatenate([x2_top, x2_bot], axis=0), 0.0)
    x3 = jnp.maximum(dot(x2, l1w_ref[...]) + l1b_ref[...][None, :], 0.0)
    x4 = jnp.maximum(dot(x3, l2w_ref[...]) + l2b_ref[...][None, :], 0.0)
    out_ref[...] = dot(x4, l3w_ref[...]) + l3b_ref[...][None, :]


def kernel(g, features, weight, edge_weight, W1, b1, lin1_W, lin1_b,
           lin2_W, lin2_b, lin3_W, lin3_b):
    a0_flat, a1_flat, deg_flat = _sc_build(
        g.reshape(2, E // 128, 128), edge_weight.reshape(E // 128, 128))
    ap0 = a0_flat.reshape(8, 64, 8, 128)
    ap1 = a1_flat.reshape(8, 64, 8, 128)
    degs = deg_flat.reshape(64, 128)
    wbf = weight.astype(jnp.bfloat16).reshape(8, 256, N)
    feat3 = features.reshape(8, 128, -1)
    w1c = W1.astype(jnp.bfloat16).reshape(256, 8, 256).transpose(1, 0, 2)

    anyspec = pl.BlockSpec(memory_space=pl.ANY)
    out = pl.pallas_call(
        _tc_body,
        in_specs=[
            anyspec,
            anyspec,
            pl.BlockSpec((64, 128), lambda: (0, 0)),
            anyspec,
            anyspec,
            pl.BlockSpec((2 * N,), lambda: (0,)),
            anyspec,
            pl.BlockSpec((N, 64), lambda: (0, 0)),
            pl.BlockSpec((64,), lambda: (0,)),
            pl.BlockSpec((64, 16), lambda: (0, 0)),
            pl.BlockSpec((16,), lambda: (0,)),
            pl.BlockSpec((16, 16), lambda: (0, 0)),
            pl.BlockSpec((16,), lambda: (0,)),
        ],
        out_specs=pl.BlockSpec((N, 16), lambda: (0, 0)),
        out_shape=jax.ShapeDtypeStruct((N, 16), jnp.float32),
        scratch_shapes=[
            pltpu.VMEM((8, 64, 8, 128), jnp.float32),
            pltpu.VMEM((8, 64, 8, 128), jnp.float32),
            pltpu.VMEM((8, 128, 256), jnp.float32),
            pltpu.VMEM((8, 256, 256), jnp.bfloat16),
            pltpu.VMEM((8, 256, N), jnp.bfloat16),
            pltpu.SemaphoreType.DMA((6,)),
        ],
    )(ap0, ap1, degs, feat3, w1c, b1, wbf,
      lin1_W, lin1_b, lin2_W, lin2_b, lin3_W, lin3_b)
    return out
